# Initial kernel scaffold; baseline (speedup 1.0000x reference)
import jax
import jax.numpy as jnp
from jax.experimental import pallas as pl


def kernel(x, sadj, fadj, params):
    raise NotImplementedError("write your pallas kernel here")


# speedup vs baseline: 1.0791x; 1.0791x over previous
import jax
import jax.numpy as jnp
from jax.experimental import pallas as pl
from jax.experimental.pallas import tpu as pltpu

_MBLK = 400


def _lsm(t):
    m = jnp.max(t, axis=1, keepdims=True)
    e = t - m
    return e - jnp.log(jnp.sum(jnp.exp(e), axis=1, keepdims=True))


def _dot(a, b):
    return jnp.dot(a, b, preferred_element_type=jnp.float32)


def _stage0_kernel(x_ref, wi_ref, bi_ref, wd_ref, bd_ref, w11_ref, w12_ref,
                   xd_ref, r1_ref, r2_ref):
    x1 = _dot(x_ref[...], wi_ref[...]) + bi_ref[...]
    xd_ref[...] = _dot(x1, wd_ref[...]) + bd_ref[...]
    r1_ref[...] = _dot(x1, w11_ref[...]).astype(jnp.bfloat16)
    r2_ref[...] = _dot(x1, w12_ref[...]).astype(jnp.bfloat16)


def _p1_kernel(adj_ref, r_ref, b1_ref, w2_ref, adjb_ref, u_ref):
    ab = adj_ref[...].astype(jnp.bfloat16)
    adjb_ref[...] = ab
    h = jnp.maximum(_dot(ab, r_ref[...]) + b1_ref[...], 0.0)
    u_ref[...] = _dot(h, w2_ref[...]).astype(jnp.bfloat16)


def _p2_kernel(adj_ref, u_ref, b2_ref, xd_ref, cw1_ref, emb_ref, v_ref):
    emb = _dot(adj_ref[...], u_ref[...]) + b2_ref[...]
    emb_ref[...] = emb
    embc = 0.85 * emb + 0.15 * xd_ref[...]
    v_ref[...] = _dot(embc, cw1_ref[...]).astype(jnp.bfloat16)


def _p3_kernel(adj_ref, v_ref, cb1_ref, cw2_ref, w_ref):
    h = jnp.maximum(_dot(adj_ref[...], v_ref[...]) + cb1_ref[...], 0.0)
    w_ref[...] = _dot(h, cw2_ref[...]).astype(jnp.bfloat16)


def _p4_kernel(adj_ref, w_ref, cb2_ref, com_ref):
    com_ref[...] = _dot(adj_ref[...], w_ref[...]) + cb2_ref[...]


def _final_kernel(e1_ref, e2_ref, c1_ref, c2_ref, avg_wa_ref, avg_wb_ref,
                  avg_b_ref, l1w_ref, l1b_ref, l2w_ref, l2b_ref, l3w_ref,
                  l3b_ref, l4wa_ref, l4wb_ref, l4b_ref,
                  L_ref, at_ref, af_ref, ac_ref):
    e1 = e1_ref[...]
    e2 = e2_ref[...]
    xcom = (_dot(c1_ref[...], avg_wa_ref[...]) +
            _dot(c2_ref[...], avg_wb_ref[...]) + avg_b_ref[...])
    xt = _lsm(jnp.tanh(_dot(e1, l1w_ref[...]) + l1b_ref[...]))
    xf = _lsm(jnp.tanh(_dot(e2, l2w_ref[...]) + l2b_ref[...]))
    xc = _lsm(jnp.tanh(_dot(xcom, l3w_ref[...]) + l3b_ref[...]))
    ztc = xt * e1 + xc * xcom
    zfc = xf * e2 + xc * xcom
    L_ref[...] = _lsm(_dot(ztc, l4wa_ref[...]) + _dot(zfc, l4wb_ref[...]) +
                      l4b_ref[...])
    at_ref[...] = xt
    af_ref[...] = xf
    ac_ref[...] = xc


def _row_spec(w):
    return pl.BlockSpec((_MBLK, w), lambda m: (m, 0))


def _full_spec(shape):
    nd = len(shape)
    return pl.BlockSpec(shape, lambda m, _n=nd: (0,) * _n)


def _big_pass(kernel_fn, adj, rhs, extras, extra_specs, out_shapes, out_specs):
    n = adj.shape[0]
    return pl.pallas_call(
        kernel_fn,
        grid=(n // _MBLK,),
        in_specs=[
            pl.BlockSpec((_MBLK, n), lambda m: (m, 0)),
            _full_spec(rhs.shape),
        ] + list(extra_specs),
        out_specs=list(out_specs),
        out_shape=list(out_shapes),
        compiler_params=pltpu.CompilerParams(
            dimension_semantics=("arbitrary",)),
    )(adj, rhs, *extras)


def _branch(adj, r, b1, w2, b2, xd, cw1, cb1, cw2, cb2):
    n = adj.shape[0]
    h2 = w2.shape[1]
    f32 = jnp.float32
    bf16 = jnp.bfloat16

    adjb, u = _big_pass(
        _p1_kernel, adj, r,
        extras=(b1, w2),
        extra_specs=(_full_spec(b1.shape), _full_spec(w2.shape)),
        out_shapes=(jax.ShapeDtypeStruct((n, n), bf16),
                    jax.ShapeDtypeStruct((n, h2), bf16)),
        out_specs=(pl.BlockSpec((_MBLK, n), lambda m: (m, 0)),
                   _row_spec(h2)))

    emb, v = _big_pass(
        _p2_kernel, adjb, u,
        extras=(b2, xd, cw1),
        extra_specs=(_full_spec(b2.shape), _row_spec(h2),
                     _full_spec(cw1.shape)),
        out_shapes=(jax.ShapeDtypeStruct((n, h2), f32),
                    jax.ShapeDtypeStruct((n, h2), bf16)),
        out_specs=(_row_spec(h2), _row_spec(h2)))

    (w,) = _big_pass(
        _p3_kernel, adjb, v,
        extras=(cb1, cw2),
        extra_specs=(_full_spec(cb1.shape), _full_spec(cw2.shape)),
        out_shapes=(jax.ShapeDtypeStruct((n, h2), bf16),),
        out_specs=(_row_spec(h2),))

    (com,) = _big_pass(
        _p4_kernel, adjb, w,
        extras=(cb2,),
        extra_specs=(_full_spec(cb2.shape),),
        out_shapes=(jax.ShapeDtypeStruct((n, h2), f32),),
        out_specs=(_row_spec(h2),))

    return emb, com


def kernel(x, sadj, fadj, params):
    p = params
    n, nfeat = x.shape
    nhid1 = p['mlp_init_W'].shape[1]
    nhid2 = p['mlp_dim_W'].shape[1]
    f32 = jnp.float32
    bf16 = jnp.bfloat16

    def b2d(name):
        return p[name].reshape(1, -1)

    mb = n // _MBLK

    xd, r1, r2 = pl.pallas_call(
        _stage0_kernel,
        grid=(mb,),
        in_specs=[
            pl.BlockSpec((_MBLK, nfeat), lambda m: (m, 0)),
            _full_spec((nfeat, nhid1)),
            _full_spec((1, nhid1)),
            _full_spec((nhid1, nhid2)),
            _full_spec((1, nhid2)),
            _full_spec((nhid1, nhid1)),
            _full_spec((nhid1, nhid1)),
        ],
        out_specs=[
            _row_spec(nhid2),
            _row_spec(nhid1),
            _row_spec(nhid1),
        ],
        out_shape=[
            jax.ShapeDtypeStruct((n, nhid2), f32),
            jax.ShapeDtypeStruct((n, nhid1), bf16),
            jax.ShapeDtypeStruct((n, nhid1), bf16),
        ],
    )(x, p['mlp_init_W'], b2d('mlp_init_b'), p['mlp_dim_W'],
      b2d('mlp_dim_b'), p['s1_W1'], p['s2_W1'])

    emb1, com1 = _branch(sadj, r1, b2d('s1_b1'), p['s1_W2'], b2d('s1_b2'),
                         xd, p['c_W1'], b2d('c_b1'), p['c_W2'], b2d('c_b2'))
    emb2, com2 = _branch(fadj, r2, b2d('s2_b1'), p['s2_W2'], b2d('s2_b2'),
                         xd, p['c_W1'], b2d('c_b1'), p['c_W2'], b2d('c_b2'))

    L, at, af, ac = pl.pallas_call(
        _final_kernel,
        grid=(mb,),
        in_specs=[
            _row_spec(nhid2), _row_spec(nhid2),
            _row_spec(nhid2), _row_spec(nhid2),
            _full_spec((nhid2, nhid2)), _full_spec((nhid2, nhid2)),
            _full_spec((1, nhid2)),
            _full_spec((nhid2, nhid2)), _full_spec((1, nhid2)),
            _full_spec((nhid2, nhid2)), _full_spec((1, nhid2)),
            _full_spec((nhid2, nhid2)), _full_spec((1, nhid2)),
            _full_spec((nhid2, nhid2)), _full_spec((nhid2, nhid2)),
            _full_spec((1, nhid2)),
        ],
        out_specs=[_row_spec(nhid2)] * 4,
        out_shape=[jax.ShapeDtypeStruct((n, nhid2), f32)] * 4,
    )(emb1, emb2, com1, com2,
      p['avg_W'][:nhid2], p['avg_W'][nhid2:], b2d('avg_b'),
      p['l1_W'], b2d('l1_b'), p['l2_W'], b2d('l2_b'),
      p['l3_W'], b2d('l3_b'),
      p['l4_W'][:nhid2], p['l4_W'][nhid2:], b2d('l4_b'))

    return (L, emb1, com1, com2, emb2, emb2, (at, af, ac, ac))
